# baseline (speedup 1.0000x reference)
import functools

import jax
import jax.numpy as jnp
from jax import lax
from jax.experimental import pallas as pl
from jax.experimental.pallas import tpu as pltpu
from jax.experimental.pallas import tpu_sc as plsc

_LANES = 16


@functools.lru_cache(maxsize=None)
def _make_sc_kernel(batch: int, hist: int):
    info = plsc.get_sparse_core_info()
    nw = info.num_cores * info.num_subcores
    assert batch % (nw * _LANES) == 0
    rows_w = batch // nw
    groups = rows_w // _LANES
    mesh = plsc.VectorSubcoreMesh(core_axis_name="c", subcore_axis_name="s")

    @functools.partial(
        pl.kernel,
        out_type=jax.ShapeDtypeStruct((batch,), jnp.float32),
        mesh=mesh,
        scratch_types=[
            pltpu.VMEM((hist, rows_w // 4), jnp.int32),
            pltpu.VMEM((hist, rows_w // 4), jnp.int32),
            pltpu.VMEM((hist, rows_w // 4), jnp.int32),
            pltpu.VMEM((hist, rows_w // 4), jnp.int32),
            pltpu.VMEM((rows_w,), jnp.float32),
            pltpu.VMEM((32,), jnp.float32),
            pltpu.SemaphoreType.DMA((4,)),
        ],
        compiler_params=pltpu.CompilerParams(
            needs_layout_passes=False,
            use_tc_tiling_on_sc=True,
        ),
    )
    def kern(idsT_hbm, par_hbm, out_hbm, ids_a, ids_b, ids_c, ids_d,
             out_v, par_v, sems):
        wid = lax.axis_index("s") * info.num_cores + lax.axis_index("c")
        base = wid * rows_w
        quarter = rows_w // 4
        bufs = (ids_a, ids_b, ids_c, ids_d)
        cps = [
            pltpu.make_async_copy(
                idsT_hbm.at[:, pl.ds(base + i * quarter, quarter)],
                bufs[i], sems.at[i])
            for i in range(4)
        ]
        for cp in cps:
            cp.start()
        pltpu.sync_copy(par_hbm, par_v)

        p0 = par_v[pl.ds(0, _LANES)]
        p1 = par_v[pl.ds(_LANES, _LANES)]

        def s_of(v):
            acc = p0[4 * v] * p0[12]
            for d in range(1, 4):
                acc = acc + p0[4 * v + d] * p0[12 + d]
            return acc

        s0, s1, s2 = s_of(0), s_of(1), s_of(2)
        bias = p1[0]
        beta = s1 - s0
        gamma = 0.5 * (s2 - 2.0 * s1 + s0)
        inv_h = 1.0 / hist

        def make_group_body(buf, out_off):
            def group_body(g, _):
                col = g * _LANES
                acc1 = jnp.zeros((_LANES,), jnp.int32)
                acc2 = jnp.zeros((_LANES,), jnp.int32)
                for j in range(hist):
                    v = buf[j, pl.ds(col, _LANES)]
                    acc1 = acc1 + v
                    acc2 = acc2 + v * v
                f1 = acc1.astype(jnp.float32)
                f2 = acc2.astype(jnp.float32)
                logit = s0 + (beta * f1 + gamma * (f2 - f1)) * inv_h + bias
                out_v[pl.ds(out_off + col, _LANES)] = (
                    1.0 / (1.0 + jnp.exp(-logit)))
                return _
            return group_body

        for i in range(4):
            cps[i].wait()
            lax.fori_loop(0, groups // 4,
                          make_group_body(bufs[i], i * quarter), None)
        pltpu.sync_copy(out_v, out_hbm.at[pl.ds(base, rows_w)])

    return kern


def kernel(color_ids, table, W, b):
    batch, hist = color_ids.shape
    params = jnp.concatenate([
        table.reshape(-1).astype(jnp.float32),
        W.reshape(-1).astype(jnp.float32),
        b.reshape(-1).astype(jnp.float32),
        jnp.zeros((15,), jnp.float32),
    ])
    ids_t = color_ids.astype(jnp.int32).T
    out = _make_sc_kernel(batch, hist)(ids_t, params)
    return out.reshape(batch, 1)


# speedup vs baseline: 1.0136x; 1.0136x over previous
import functools

import jax
import jax.numpy as jnp
from jax import lax
from jax.experimental import pallas as pl
from jax.experimental.pallas import tpu as pltpu
from jax.experimental.pallas import tpu_sc as plsc

_LANES = 16


@functools.lru_cache(maxsize=None)
def _make_sc_kernel(batch: int, hist: int):
    info = plsc.get_sparse_core_info()
    nw = info.num_cores * info.num_subcores
    assert batch % (nw * _LANES) == 0
    rows_w = batch // nw
    groups = rows_w // _LANES
    mesh = plsc.VectorSubcoreMesh(core_axis_name="c", subcore_axis_name="s")

    @functools.partial(
        pl.kernel,
        out_type=jax.ShapeDtypeStruct((batch,), jnp.float32),
        mesh=mesh,
        scratch_types=[
            pltpu.VMEM((hist, rows_w // 2), jnp.int32),
            pltpu.VMEM((hist, rows_w // 2), jnp.int32),
            pltpu.VMEM((rows_w,), jnp.float32),
            pltpu.VMEM((32,), jnp.float32),
            pltpu.SemaphoreType.DMA((2,)),
        ],
        compiler_params=pltpu.CompilerParams(
            needs_layout_passes=False,
            use_tc_tiling_on_sc=True,
        ),
    )
    def kern(idsT_hbm, par_hbm, out_hbm, ids_a, ids_b, out_v, par_v, sems):
        wid = lax.axis_index("s") * info.num_cores + lax.axis_index("c")
        base = wid * rows_w
        half = rows_w // 2
        cp_a = pltpu.make_async_copy(
            idsT_hbm.at[:, pl.ds(base, half)], ids_a, sems.at[0])
        cp_b = pltpu.make_async_copy(
            idsT_hbm.at[:, pl.ds(base + half, half)], ids_b, sems.at[1])
        cp_a.start()
        cp_b.start()
        pltpu.sync_copy(par_hbm, par_v)

        p0 = par_v[pl.ds(0, _LANES)]
        p1 = par_v[pl.ds(_LANES, _LANES)]

        def s_of(v):
            acc = p0[4 * v] * p0[12]
            for d in range(1, 4):
                acc = acc + p0[4 * v + d] * p0[12 + d]
            return acc

        s0, s1, s2 = s_of(0), s_of(1), s_of(2)
        bias = p1[0]
        beta = s1 - s0
        gamma = 0.5 * (s2 - 2.0 * s1 + s0)
        inv_h = 1.0 / hist

        def make_group_body(buf, out_off):
            def group_body(g, _):
                col = g * _LANES
                acc1 = jnp.zeros((_LANES,), jnp.int32)
                acc2 = jnp.zeros((_LANES,), jnp.int32)
                for j in range(hist):
                    v = buf[j, pl.ds(col, _LANES)]
                    acc1 = acc1 + v
                    acc2 = acc2 + v * v
                f1 = acc1.astype(jnp.float32)
                f2 = acc2.astype(jnp.float32)
                logit = s0 + (beta * f1 + gamma * (f2 - f1)) * inv_h + bias
                out_v[pl.ds(out_off + col, _LANES)] = (
                    1.0 / (1.0 + jnp.exp(-logit)))
                return _
            return group_body

        cp_a.wait()
        lax.fori_loop(0, groups // 2, make_group_body(ids_a, 0), None)
        cp_b.wait()
        lax.fori_loop(0, groups // 2, make_group_body(ids_b, half), None)
        pltpu.sync_copy(out_v, out_hbm.at[pl.ds(base, rows_w)])

    return kern


def kernel(color_ids, table, W, b):
    batch, hist = color_ids.shape
    params = jnp.concatenate([
        table.reshape(-1).astype(jnp.float32),
        W.reshape(-1).astype(jnp.float32),
        b.reshape(-1).astype(jnp.float32),
        jnp.zeros((15,), jnp.float32),
    ])
    ids_t = color_ids.astype(jnp.int32).T
    out = _make_sc_kernel(batch, hist)(ids_t, params)
    return out.reshape(batch, 1)
